# baseline (speedup 1.0000x reference)
import functools
import math

import jax
import jax.numpy as jnp
from jax import lax
from jax.experimental import pallas as pl
from jax.experimental.pallas import tpu as pltpu

_HALF_LOG_2PI = 0.5 * math.log(2.0 * math.pi)


def _body(z_ref, mu_ref, lv_ref, w_ref, out_ref, a_s, b_s, c_s):
    K = mu_ref.shape[0]
    mu = mu_ref[...]
    lv = lv_ref[...]
    wv = w_ref[...]
    wmax = jnp.max(wv)
    logw = wv - wmax - jnp.log(jnp.sum(jnp.exp(wv - wmax)))
    prec = jnp.exp(-lv)
    A = (-_HALF_LOG_2PI) - 0.5 * lv - 0.5 * prec * mu * mu + logw
    Bc = prec * mu
    Cc = -0.5 * prec
    a_s[...] = jnp.concatenate([A, A], axis=1)
    b_s[...] = jnp.concatenate([Bc, Bc], axis=1)
    c_s[...] = jnp.concatenate([Cc, Cc], axis=1)

    zb = z_ref[...]
    z2 = zb * zb

    def affine(k):
        a = a_s[pl.ds(k, 1), :]
        b = b_s[pl.ds(k, 1), :]
        c = c_s[pl.ds(k, 1), :]
        return a + b * zb + c * z2

    def pass1(k, m):
        return jnp.maximum(m, affine(k))

    m = lax.fori_loop(0, K, pass1, jnp.full(zb.shape, -jnp.inf, jnp.float32))

    def pass2(k, s):
        return s + jnp.exp(affine(k) - m)

    s = lax.fori_loop(0, K, pass2, jnp.zeros(zb.shape, jnp.float32))
    out_ref[...] = m + jnp.log(s)


@jax.jit
def kernel(z, means, logvars, w):
    B, L = z.shape
    K = means.shape[0]
    zr = z.reshape(B // 2, 2 * L)
    w2 = w.reshape(K, 1)
    BB = 256
    grid = (zr.shape[0] // BB,)
    out = pl.pallas_call(
        _body,
        grid=grid,
        in_specs=[
            pl.BlockSpec((BB, 2 * L), lambda i: (i, 0)),
            pl.BlockSpec((K, L), lambda i: (0, 0)),
            pl.BlockSpec((K, L), lambda i: (0, 0)),
            pl.BlockSpec((K, 1), lambda i: (0, 0)),
        ],
        out_specs=pl.BlockSpec((BB, 2 * L), lambda i: (i, 0)),
        out_shape=jax.ShapeDtypeStruct(zr.shape, jnp.float32),
        scratch_shapes=[
            pltpu.VMEM((K, 2 * L), jnp.float32),
            pltpu.VMEM((K, 2 * L), jnp.float32),
            pltpu.VMEM((K, 2 * L), jnp.float32),
        ],
    )(zr, means, logvars, w2)
    return out.reshape(B, L)


# speedup vs baseline: 1.1616x; 1.1616x over previous
import functools
import math

import jax
import jax.numpy as jnp
from jax import lax
from jax.experimental import pallas as pl
from jax.experimental.pallas import tpu as pltpu

_HALF_LOG_2PI = 0.5 * math.log(2.0 * math.pi)
_LG = 8


def _body(zt_ref, mu_ref, lv_ref, w_ref, out_ref, p_s):
    K, L = mu_ref.shape
    Bb = zt_ref.shape[1]
    mu = mu_ref[...]
    lv = lv_ref[...]
    wv = w_ref[...]
    wmax = jnp.max(wv)
    logw = wv - wmax - jnp.log(jnp.sum(jnp.exp(wv - wmax)))
    prec = jnp.exp(-lv)
    p_s[0:K, :] = (-_HALF_LOG_2PI) - 0.5 * lv - 0.5 * prec * mu * mu + logw
    p_s[K:2 * K, :] = prec * mu
    p_s[2 * K:3 * K, :] = -0.5 * prec

    pmat = p_s[...]
    row_i = lax.broadcasted_iota(jnp.int32, (L, _LG), 0)
    col_i = lax.broadcasted_iota(jnp.int32, (L, _LG), 1)

    def lgroup(g, _):
        oh = jnp.where(row_i == g * _LG + col_i, 1.0, 0.0).astype(jnp.float32)
        pc = lax.dot_general(pmat, oh, (((1,), (0,)), ((), ())),
                             precision=lax.Precision.HIGHEST)
        for j in range(_LG):
            col = pc[:, j:j + 1]
            a = col[0:K]
            b = col[K:2 * K]
            c = col[2 * K:3 * K]
            l = g * _LG + j
            zrow = zt_ref[pl.ds(l, 1), :]
            z2 = zrow * zrow
            t = a + b * zrow + c * z2
            m = jnp.max(t, axis=0, keepdims=True)
            s = jnp.sum(jnp.exp(t - m), axis=0, keepdims=True)
            out_ref[pl.ds(l, 1), :] = m + jnp.log(s)
        return 0

    lax.fori_loop(0, L // _LG, lgroup, 0)


@jax.jit
def kernel(z, means, logvars, w):
    B, L = z.shape
    K = means.shape[0]
    zt = z.T
    w2 = w.reshape(K, 1)
    Bb = 512
    grid = (B // Bb,)
    out = pl.pallas_call(
        _body,
        grid=grid,
        in_specs=[
            pl.BlockSpec((L, Bb), lambda i: (0, i)),
            pl.BlockSpec((K, L), lambda i: (0, 0)),
            pl.BlockSpec((K, L), lambda i: (0, 0)),
            pl.BlockSpec((K, 1), lambda i: (0, 0)),
        ],
        out_specs=pl.BlockSpec((L, Bb), lambda i: (0, i)),
        out_shape=jax.ShapeDtypeStruct((L, B), jnp.float32),
        scratch_shapes=[
            pltpu.VMEM((3 * K, L), jnp.float32),
        ],
    )(zt, means, logvars, w2)
    return out.T
